# Initial kernel scaffold; baseline (speedup 1.0000x reference)
import jax
import jax.numpy as jnp
from jax.experimental import pallas as pl


def kernel(input, index):
    raise NotImplementedError("write your pallas kernel here")


# speedup vs baseline: 1.0853x; 1.0853x over previous
import functools

import jax
import jax.numpy as jnp
from jax import lax
from jax.experimental import pallas as pl
from jax.experimental.pallas import tpu as pltpu
from jax.experimental.pallas import tpu_sc as plsc

B = 1024
R = 200
I = 50
C = 128
IN_W = R * C
OUT_W = I * C
L = 16

_info = plsc.get_sparse_core_info()
_NC, _NS = _info.num_cores, _info.num_subcores
NW = _NC * _NS
BPW = B // NW

_mesh = plsc.VectorSubcoreMesh(core_axis_name="c", subcore_axis_name="s")


@functools.partial(
    pl.kernel,
    mesh=_mesh,
    out_type=jax.ShapeDtypeStruct((B, OUT_W), jnp.float32),
    scratch_types=[
        pltpu.VMEM((IN_W,), jnp.float32),
        pltpu.VMEM((OUT_W,), jnp.int32),
        pltpu.VMEM((OUT_W,), jnp.float32),
    ],
    compiler_params=pltpu.CompilerParams(needs_layout_passes=False),
)
def _gather_sc(in_hbm, idx_hbm, out_hbm, in_v, idx_v, out_v):
    wid = lax.axis_index("s") * _NC + lax.axis_index("c")
    lane = lax.iota(jnp.int32, 16)

    def batch_body(bi, carry):
        b = wid * BPW + bi
        pltpu.sync_copy(in_hbm.at[b], in_v)
        pltpu.sync_copy(idx_hbm.at[b], idx_v)

        def vec_body(v, c2):
            idx16 = idx_v[pl.ds(v * L, L)]
            k0 = lax.rem(v, C // L) * L
            addr = idx16 * C + (k0 + lane)
            out_v[pl.ds(v * L, L)] = plsc.load_gather(in_v, [addr])
            return c2

        lax.fori_loop(0, OUT_W // L, vec_body, 0)
        pltpu.sync_copy(out_v, out_hbm.at[b])
        return carry

    lax.fori_loop(0, BPW, batch_body, 0)


def kernel(input, index):
    inp = input.reshape(B, IN_W)
    idx = index.reshape(B, OUT_W).astype(jnp.int32)
    out = _gather_sc(inp, idx)
    return out.reshape(B, I, C)


# speedup vs baseline: 1.8437x; 1.6989x over previous
import functools

import jax
import jax.numpy as jnp
from jax import lax
from jax.experimental import pallas as pl
from jax.experimental.pallas import tpu as pltpu
from jax.experimental.pallas import tpu_sc as plsc

B = 1024
R = 200
I = 50
C = 128
IN_W = R * C
OUT_W = I * C
L = 16
G = C // L

_info = plsc.get_sparse_core_info()
_NC, _NS = _info.num_cores, _info.num_subcores
NW = _NC * _NS
BPW = B // NW

_mesh = plsc.VectorSubcoreMesh(core_axis_name="c", subcore_axis_name="s")


@functools.partial(
    pl.kernel,
    mesh=_mesh,
    out_type=jax.ShapeDtypeStruct((B, OUT_W), jnp.float32),
    scratch_types=[
        pltpu.VMEM((IN_W,), jnp.float32),
        pltpu.VMEM((IN_W,), jnp.float32),
        pltpu.VMEM((OUT_W,), jnp.int32),
        pltpu.VMEM((OUT_W,), jnp.int32),
        pltpu.VMEM((OUT_W,), jnp.float32),
        pltpu.VMEM((OUT_W,), jnp.float32),
        pltpu.SemaphoreType.DMA,
        pltpu.SemaphoreType.DMA,
        pltpu.SemaphoreType.DMA,
        pltpu.SemaphoreType.DMA,
        pltpu.SemaphoreType.DMA,
        pltpu.SemaphoreType.DMA,
    ],
    compiler_params=pltpu.CompilerParams(needs_layout_passes=False),
)
def _gather_sc(in_hbm, idx_hbm, out_hbm,
               in_v0, in_v1, idx_v0, idx_v1, out_v0, out_v1,
               in_s0, in_s1, idx_s0, idx_s1, out_s0, out_s1):
    wid = lax.axis_index("s") * _NC + lax.axis_index("c")
    b0 = wid * BPW
    lane = lax.iota(jnp.int32, L)
    lanes = [lane + g * L for g in range(G)]

    slots = (
        (in_v0, idx_v0, out_v0, in_s0, idx_s0, out_s0),
        (in_v1, idx_v1, out_v1, in_s1, idx_s1, out_s1),
    )

    def start_in(b, slot):
        in_v, idx_v, _, in_s, idx_s, _ = slots[slot]
        pltpu.async_copy(in_hbm.at[b], in_v, in_s)
        pltpu.async_copy(idx_hbm.at[b], idx_v, idx_s)

    def wait_in(b, slot):
        in_v, idx_v, _, in_s, idx_s, _ = slots[slot]
        pltpu.make_async_copy(in_hbm.at[b], in_v, in_s).wait()
        pltpu.make_async_copy(idx_hbm.at[b], idx_v, idx_s).wait()

    def wait_out(b, slot):
        _, _, out_v, _, _, out_s = slots[slot]
        pltpu.make_async_copy(out_v, out_hbm.at[b], out_s).wait()

    def compute(slot):
        in_v, idx_v, out_v, _, _, _ = slots[slot]

        def row(i, carry):
            base = i * C
            for g in range(G):
                off = base + g * L
                idx16 = idx_v[pl.ds(off, L)]
                addr = idx16 * C + lanes[g]
                out_v[pl.ds(off, L)] = plsc.load_gather(in_v, [addr])
            return carry

        lax.fori_loop(0, I, row, 0)

    def start_out(b, slot):
        _, _, out_v, _, _, out_s = slots[slot]
        pltpu.async_copy(out_v, out_hbm.at[b], out_s)

    start_in(b0, 0)

    def pair_body(p, carry):
        bi0 = 2 * p
        for slot in range(2):
            b = b0 + bi0 + slot
            nxt = bi0 + slot + 1

            @pl.when(nxt < BPW)
            def _():
                start_in(b0 + nxt, 1 - slot)

            @pl.when(p > 0)
            def _():
                wait_out(b - 2, slot)

            wait_in(b, slot)
            compute(slot)
            start_out(b, slot)
        return carry

    lax.fori_loop(0, BPW // 2, pair_body, 0)
    wait_out(b0 + BPW - 2, 0)
    wait_out(b0 + BPW - 1, 1)


def kernel(input, index):
    inp = input.reshape(B, IN_W)
    idx = index.reshape(B, OUT_W).astype(jnp.int32)
    out = _gather_sc(inp, idx)
    return out.reshape(B, I, C)


# speedup vs baseline: 3.3388x; 1.8109x over previous
import functools

import jax
import jax.numpy as jnp
from jax import lax
from jax.experimental import pallas as pl
from jax.experimental.pallas import tpu as pltpu
from jax.experimental.pallas import tpu_sc as plsc

B = 1024
R = 200
I = 50
C = 128
L = 16
G = C // L

_info = plsc.get_sparse_core_info()
_NC, _NS = _info.num_cores, _info.num_subcores
NW = _NC * _NS
BPW = B // NW

_mesh = plsc.VectorSubcoreMesh(core_axis_name="c", subcore_axis_name="s")


@functools.partial(
    pl.kernel,
    mesh=_mesh,
    out_type=jax.ShapeDtypeStruct((B, I, C), jnp.float32),
    scratch_types=[
        pltpu.VMEM((R, C), jnp.float32),
        pltpu.VMEM((R, C), jnp.float32),
        pltpu.VMEM((I, C), jnp.int32),
        pltpu.VMEM((I, C), jnp.int32),
        pltpu.VMEM((I, C), jnp.float32),
        pltpu.VMEM((I, C), jnp.float32),
        pltpu.SemaphoreType.DMA,
        pltpu.SemaphoreType.DMA,
        pltpu.SemaphoreType.DMA,
        pltpu.SemaphoreType.DMA,
        pltpu.SemaphoreType.DMA,
        pltpu.SemaphoreType.DMA,
    ],
    compiler_params=pltpu.CompilerParams(
        needs_layout_passes=False,
        use_tc_tiling_on_sc=True,
    ),
)
def _gather_sc(in_hbm, idx_hbm, out_hbm,
               in_v0, in_v1, idx_v0, idx_v1, out_v0, out_v1,
               in_s0, in_s1, idx_s0, idx_s1, out_s0, out_s1):
    wid = lax.axis_index("s") * _NC + lax.axis_index("c")
    b0 = wid * BPW
    lane = lax.iota(jnp.int32, L)
    lanes = [lane + g * L for g in range(G)]

    slots = (
        (in_v0, idx_v0, out_v0, in_s0, idx_s0, out_s0),
        (in_v1, idx_v1, out_v1, in_s1, idx_s1, out_s1),
    )

    def start_in(b, slot):
        in_v, idx_v, _, in_s, idx_s, _ = slots[slot]
        pltpu.async_copy(in_hbm.at[b], in_v, in_s)
        pltpu.async_copy(idx_hbm.at[b], idx_v, idx_s)

    def wait_in(b, slot):
        in_v, idx_v, _, in_s, idx_s, _ = slots[slot]
        pltpu.make_async_copy(in_hbm.at[b], in_v, in_s).wait()
        pltpu.make_async_copy(idx_hbm.at[b], idx_v, idx_s).wait()

    def wait_out(b, slot):
        _, _, out_v, _, _, out_s = slots[slot]
        pltpu.make_async_copy(out_v, out_hbm.at[b], out_s).wait()

    def start_out(b, slot):
        _, _, out_v, _, _, out_s = slots[slot]
        pltpu.async_copy(out_v, out_hbm.at[b], out_s)

    def compute(slot):
        in_v, idx_v, out_v, _, _, _ = slots[slot]

        def row(i, carry):
            for g in range(G):
                idx16 = idx_v[i, pl.ds(g * L, L)]
                out_v[i, pl.ds(g * L, L)] = plsc.load_gather(
                    in_v, [idx16, lanes[g]]
                )
            return carry

        lax.fori_loop(0, I, row, 0)

    start_in(b0, 0)

    def pair_body(p, carry):
        bi0 = 2 * p
        for slot in range(2):
            b = b0 + bi0 + slot
            nxt = bi0 + slot + 1

            @pl.when(nxt < BPW)
            def _():
                start_in(b0 + nxt, 1 - slot)

            @pl.when(p > 0)
            def _():
                wait_out(b - 2, slot)

            wait_in(b, slot)
            compute(slot)
            start_out(b, slot)
        return carry

    lax.fori_loop(0, BPW // 2, pair_body, 0)
    wait_out(b0 + BPW - 2, 0)
    wait_out(b0 + BPW - 1, 1)


def kernel(input, index):
    return _gather_sc(input, index.astype(jnp.int32))


# speedup vs baseline: 5.4783x; 1.6408x over previous
import functools

import jax
import jax.numpy as jnp
from jax import lax
from jax.experimental import pallas as pl
from jax.experimental.pallas import tpu as pltpu
from jax.experimental.pallas import tpu_sc as plsc

B = 1024
R = 200
RH = 96
I = 50
C = 128
L = 16
G = C // L

_info = plsc.get_sparse_core_info()
_NC, _NS = _info.num_cores, _info.num_subcores
NW = _NC * _NS
BPW = B // NW

_mesh = plsc.VectorSubcoreMesh(core_axis_name="c", subcore_axis_name="s")


@functools.partial(
    pl.kernel,
    mesh=_mesh,
    out_type=jax.ShapeDtypeStruct((I, B, C), jnp.float32),
    scratch_types=[
        pltpu.VMEM((R, C), jnp.float32),
        pltpu.VMEM((R, C), jnp.float32),
        pltpu.VMEM((I, C), jnp.int32),
        pltpu.VMEM((I, C), jnp.int32),
        pltpu.VMEM((I, C), jnp.float32),
        pltpu.VMEM((I, C), jnp.float32),
        pltpu.SemaphoreType.DMA,
        pltpu.SemaphoreType.DMA,
        pltpu.SemaphoreType.DMA,
        pltpu.SemaphoreType.DMA,
        pltpu.SemaphoreType.DMA,
        pltpu.SemaphoreType.DMA,
        pltpu.SemaphoreType.DMA,
        pltpu.SemaphoreType.DMA,
    ],
    compiler_params=pltpu.CompilerParams(
        needs_layout_passes=False,
        use_tc_tiling_on_sc=True,
    ),
)
def _gather_sc(in_hbm, idx_hbm, out_hbm,
               in_v0, in_v1, idx_v0, idx_v1, out_v0, out_v1,
               ina_s0, ina_s1, inb_s0, inb_s1,
               idx_s0, idx_s1, out_s0, out_s1):
    wid = lax.axis_index("s") * _NC + lax.axis_index("c")
    b0 = wid * BPW
    lane = lax.iota(jnp.int32, L)
    lanes = [lane + g * L for g in range(G)]

    slots = (
        (in_v0, idx_v0, out_v0, ina_s0, inb_s0, idx_s0, out_s0),
        (in_v1, idx_v1, out_v1, ina_s1, inb_s1, idx_s1, out_s1),
    )

    def start_in(b, slot):
        in_v, idx_v, _, ina_s, inb_s, idx_s, _ = slots[slot]
        pltpu.async_copy(
            in_hbm.at[b, pl.ds(0, RH)], in_v.at[pl.ds(0, RH)], ina_s
        )
        pltpu.async_copy(
            in_hbm.at[b, pl.ds(RH, R - RH)], in_v.at[pl.ds(RH, R - RH)], inb_s
        )
        pltpu.async_copy(idx_hbm.at[:, b], idx_v, idx_s)

    def wait_in(b, slot):
        in_v, idx_v, _, ina_s, inb_s, idx_s, _ = slots[slot]
        pltpu.make_async_copy(
            in_hbm.at[b, pl.ds(0, RH)], in_v.at[pl.ds(0, RH)], ina_s
        ).wait()
        pltpu.make_async_copy(
            in_hbm.at[b, pl.ds(RH, R - RH)], in_v.at[pl.ds(RH, R - RH)], inb_s
        ).wait()
        pltpu.make_async_copy(idx_hbm.at[:, b], idx_v, idx_s).wait()

    def wait_out(b, slot):
        _, _, out_v, _, _, _, out_s = slots[slot]
        pltpu.make_async_copy(out_v, out_hbm.at[:, b], out_s).wait()

    def start_out(b, slot):
        _, _, out_v, _, _, _, out_s = slots[slot]
        pltpu.async_copy(out_v, out_hbm.at[:, b], out_s)

    def compute(slot):
        in_v, idx_v, out_v, _, _, _, _ = slots[slot]

        def row(i, carry):
            for g in range(G):
                idx16 = idx_v[i, pl.ds(g * L, L)]
                out_v[i, pl.ds(g * L, L)] = plsc.load_gather(
                    in_v, [idx16, lanes[g]]
                )
            return carry

        lax.fori_loop(0, I, row, 0)

    start_in(b0, 0)

    def pair_body(p, carry):
        bi0 = 2 * p
        for slot in range(2):
            b = b0 + bi0 + slot
            nxt = bi0 + slot + 1

            @pl.when(nxt < BPW)
            def _():
                start_in(b0 + nxt, 1 - slot)

            @pl.when(p > 0)
            def _():
                wait_out(b - 2, slot)

            wait_in(b, slot)
            compute(slot)
            start_out(b, slot)
        return carry

    lax.fori_loop(0, BPW // 2, pair_body, 0)
    wait_out(b0 + BPW - 2, 0)
    wait_out(b0 + BPW - 1, 1)


def kernel(input, index):
    idx_t = index.astype(jnp.int32).transpose(1, 0, 2)
    out_t = _gather_sc(input, idx_t)
    return out_t.transpose(1, 0, 2)
